# Initial kernel scaffold; baseline (speedup 1.0000x reference)
import jax
import jax.numpy as jnp
from jax.experimental import pallas as pl


def kernel(inputs, adj, H_v):
    raise NotImplementedError("write your pallas kernel here")


# speedup vs baseline: 1.1837x; 1.1837x over previous
import jax
import jax.numpy as jnp
from jax.experimental import pallas as pl


def _fused_attn_kernel(adj_ref, x_ref, hv_ref, out_ref):
    x = x_ref[...]
    s = jnp.dot(x, hv_ref[...],
                preferred_element_type=jnp.float32)[:, 0]
    c = jnp.max(jnp.abs(s))
    a = adj_ref[...]
    e = jnp.where(a != 0.0, jnp.exp(a * s[None, :] - c), 0.0)
    denom = jnp.sum(e, axis=1, keepdims=True)
    acc = jnp.dot(e, x, preferred_element_type=jnp.float32)
    out_ref[...] = acc / denom


def kernel(inputs, adj, H_v):
    n, d = inputs.shape
    bm = 256
    grid = (n // bm,)
    return pl.pallas_call(
        _fused_attn_kernel,
        grid=grid,
        in_specs=[
            pl.BlockSpec((bm, n), lambda i: (i, 0)),
            pl.BlockSpec((n, d), lambda i: (0, 0)),
            pl.BlockSpec((d, 1), lambda i: (0, 0)),
        ],
        out_specs=pl.BlockSpec((bm, d), lambda i: (i, 0)),
        out_shape=jax.ShapeDtypeStruct((n, d), jnp.float32),
    )(adj, inputs, H_v)


# speedup vs baseline: 2.0533x; 1.7347x over previous
import jax
import jax.numpy as jnp
from jax.experimental import pallas as pl
from jax.experimental.pallas import tpu as pltpu


def _fused_attn_kernel(adj_ref, x_ref, hv_ref, out_ref, s_ref, c_ref):
    @pl.when(pl.program_id(0) == 0)
    def _prologue():
        s = jnp.dot(x_ref[...], hv_ref[...],
                    preferred_element_type=jnp.float32)
        s_ref[...] = s.T
        c_ref[0, 0] = jnp.max(jnp.abs(s))

    s = s_ref[...]
    c = c_ref[0, 0]
    a = adj_ref[...]
    e = jnp.where(a != 0.0, jnp.exp(a * s - c), 0.0)
    denom = jnp.sum(e, axis=1, keepdims=True)
    x = x_ref[...]
    acc = jnp.dot(e, x, preferred_element_type=jnp.float32)
    out_ref[...] = acc / denom


def kernel(inputs, adj, H_v):
    n, d = inputs.shape
    bm = 256
    grid = (n // bm,)
    return pl.pallas_call(
        _fused_attn_kernel,
        grid=grid,
        in_specs=[
            pl.BlockSpec((bm, n), lambda i: (i, 0)),
            pl.BlockSpec((n, d), lambda i: (0, 0)),
            pl.BlockSpec((d, 1), lambda i: (0, 0)),
        ],
        out_specs=pl.BlockSpec((bm, d), lambda i: (i, 0)),
        out_shape=jax.ShapeDtypeStruct((n, d), jnp.float32),
        scratch_shapes=[
            pltpu.VMEM((1, n), jnp.float32),
            pltpu.SMEM((1, 1), jnp.float32),
        ],
    )(adj, inputs, H_v)


# speedup vs baseline: 2.0937x; 1.0196x over previous
import jax
import jax.numpy as jnp
from jax.experimental import pallas as pl
from jax.experimental.pallas import tpu as pltpu


_LOG2E = 1.4426950408889634


def _fused_attn_kernel(adj_ref, x_ref, hv_ref, out_ref, s_ref, c_ref, xb_ref):
    @pl.when(pl.program_id(0) == 0)
    def _prologue():
        s = jnp.dot(x_ref[...], hv_ref[...],
                    preferred_element_type=jnp.float32)
        s2 = s * _LOG2E
        s_ref[...] = s2.T
        c_ref[0, 0] = jnp.max(jnp.abs(s2))
        xb_ref[...] = x_ref[...].astype(jnp.bfloat16)

    s2 = s_ref[...]
    c2 = c_ref[0, 0]
    a = adj_ref[...]
    e = jnp.where(a != 0.0, jnp.exp2(a * s2 - c2), 0.0)
    denom = jnp.sum(e, axis=1, keepdims=True)
    acc = jnp.dot(e.astype(jnp.bfloat16), xb_ref[...],
                  preferred_element_type=jnp.float32)
    out_ref[...] = acc / denom


def kernel(inputs, adj, H_v):
    n, d = inputs.shape
    bm = 256
    grid = (n // bm,)
    return pl.pallas_call(
        _fused_attn_kernel,
        grid=grid,
        in_specs=[
            pl.BlockSpec((bm, n), lambda i: (i, 0)),
            pl.BlockSpec((n, d), lambda i: (0, 0)),
            pl.BlockSpec((d, 1), lambda i: (0, 0)),
        ],
        out_specs=pl.BlockSpec((bm, d), lambda i: (i, 0)),
        out_shape=jax.ShapeDtypeStruct((n, d), jnp.float32),
        scratch_shapes=[
            pltpu.VMEM((1, n), jnp.float32),
            pltpu.SMEM((1, 1), jnp.float32),
            pltpu.VMEM((n, d), jnp.bfloat16),
        ],
    )(adj, inputs, H_v)
